# baseline (speedup 1.0000x reference)
import functools

import jax
import jax.numpy as jnp
from jax import lax
from jax.experimental import pallas as pl
from jax.experimental.pallas import tpu as pltpu
from jax.experimental.pallas import tpu_sc as plsc

B = 16384
D = 64
D2 = 128
NUM_ROWS2 = 500000
NLBL = 64

NC, NS = 2, 16
NW = NC * NS
BPW = B // NW
CH = 128
NCH = BPW // CH

_mesh = plsc.VectorSubcoreMesh(core_axis_name="c", subcore_axis_name="s")


@functools.partial(
    pl.kernel,
    mesh=_mesh,
    out_type=jax.ShapeDtypeStruct((B, D2), jnp.float32),
    scratch_types=[
        pltpu.VMEM((NCH, CH), jnp.int32),
        pltpu.VMEM((BPW, D2), jnp.float32),
        pltpu.SemaphoreType.DMA,
    ],
    compiler_params=pltpu.CompilerParams(use_tc_tiling_on_sc=False),
)
def _sc_gather(idx_hbm, table_hbm, out_hbm, idx_v, rows_v, sem):
    wid = lax.axis_index("s") * NC + lax.axis_index("c")
    base = wid * BPW
    pltpu.sync_copy(idx_hbm.at[wid], idx_v)
    copies = []
    for j in range(NCH):
        copies.append(
            pltpu.async_copy(
                table_hbm.at[idx_v.at[j]],
                rows_v.at[pl.ds(j * CH, CH)],
                sem,
            )
        )
    for c in copies:
        c.wait()
    pltpu.sync_copy(rows_v, out_hbm.at[pl.ds(base, BPW)])


def _mm_body(g_ref, p_ref, wt_ref, b_ref, o_ref):
    g = g_ref[...]
    p = p_ref[...]
    h = g[:, :D] * (1.0 - p) + g[:, D:] * p
    o_ref[...] = (
        jnp.dot(h, wt_ref[...], preferred_element_type=jnp.float32) + b_ref[...]
    )


MB = 2048


def _tc_linear(g, par, wt, b2):
    return pl.pallas_call(
        _mm_body,
        grid=(B // MB,),
        in_specs=[
            pl.BlockSpec((MB, D2), lambda i: (i, 0)),
            pl.BlockSpec((MB, 1), lambda i: (i, 0)),
            pl.BlockSpec((D, NLBL), lambda i: (0, 0)),
            pl.BlockSpec((1, NLBL), lambda i: (0, 0)),
        ],
        out_specs=pl.BlockSpec((MB, NLBL), lambda i: (i, 0)),
        out_shape=jax.ShapeDtypeStruct((B, NLBL), jnp.float32),
    )(g, par, wt, b2)


def kernel(node_seq, table, W, b):
    idx = node_seq.astype(jnp.int32)
    table2 = table.reshape(NUM_ROWS2, D2)
    hi = (idx // 2).reshape(NW, NCH, CH)
    par = (idx % 2).astype(jnp.float32).reshape(B, 1)
    g2 = _sc_gather(hi, table2)
    return _tc_linear(g2, par, W.T, b.reshape(1, NLBL))


# speedup vs baseline: 1.9801x; 1.9801x over previous
import functools

import jax
import jax.numpy as jnp
from jax import lax
from jax.experimental import pallas as pl
from jax.experimental.pallas import tpu as pltpu
from jax.experimental.pallas import tpu_sc as plsc

B = 16384
D = 64
D2 = 128
NLBL = 64
NT = 125000

NC, NS = 2, 16
NW = NC * NS
BPW = B // NW
CH = 32
NCHUNK = BPW // CH
L = 16

_mesh = plsc.VectorSubcoreMesh(core_axis_name="c", subcore_axis_name="s")


@functools.partial(
    pl.kernel,
    mesh=_mesh,
    out_type=jax.ShapeDtypeStruct((B, D2), jnp.float32),
    scratch_types=[
        pltpu.VMEM((BPW,), jnp.int32),
        pltpu.VMEM((BPW,), jnp.int32),
        pltpu.VMEM((CH, 8, D), jnp.float32),
        pltpu.VMEM((CH, D2), jnp.float32),
        pltpu.SemaphoreType.DMA,
    ],
    compiler_params=pltpu.CompilerParams(needs_layout_passes=False),
)
def _sc_gather(idx_hbm, tidx_hbm, table_hbm, out_hbm, idx_v, tidx_v,
               stage_v, rows_v, sem):
    wid = lax.axis_index("s") * NC + lax.axis_index("c")
    base = wid * BPW
    pltpu.sync_copy(idx_hbm.at[wid], idx_v)
    pltpu.sync_copy(tidx_hbm.at[wid], tidx_v)

    @pl.loop(0, NCHUNK)
    def _chunk(k):
        off = k * CH
        copies = []
        for g in range(CH // L):
            t16 = tidx_v[pl.ds(off + g * L, L)]
            for lane in range(L):
                i = g * L + lane
                copies.append(
                    pltpu.async_copy(
                        table_hbm.at[t16[lane]], stage_v.at[i], sem
                    )
                )
        for c in copies:
            c.wait()
        for g in range(CH // L):
            row16 = lax.iota(jnp.int32, L) + g * L
            idx16 = idx_v[pl.ds(off + g * L, L)]
            s16 = jnp.bitwise_and(idx16, 7)
            for c in range(D):
                c16 = jnp.full((L,), c, jnp.int32)
                val = plsc.load_gather(stage_v, [row16, s16, c16])
                plsc.store_scatter(rows_v, [row16, c16], val)
        pltpu.sync_copy(rows_v, out_hbm.at[pl.ds(base + off, CH)])


def _mm_body(g_ref, wt_ref, b_ref, o_ref):
    o_ref[...] = (
        jnp.dot(g_ref[:, :D], wt_ref[...], preferred_element_type=jnp.float32)
        + b_ref[...]
    )


MB = 2048


def _tc_linear(g, wt, b2):
    return pl.pallas_call(
        _mm_body,
        grid=(B // MB,),
        in_specs=[
            pl.BlockSpec((MB, D2), lambda i: (i, 0)),
            pl.BlockSpec((D, NLBL), lambda i: (0, 0)),
            pl.BlockSpec((1, NLBL), lambda i: (0, 0)),
        ],
        out_specs=pl.BlockSpec((MB, NLBL), lambda i: (i, 0)),
        out_shape=jax.ShapeDtypeStruct((B, NLBL), jnp.float32),
    )(g, wt, b2)


def kernel(node_seq, table, W, b):
    idx = node_seq.astype(jnp.int32)
    tbl3 = table.reshape(NT, 8, D)
    idx2 = idx.reshape(NW, BPW)
    tidx2 = (idx2 // 8).astype(jnp.int32)
    g2 = _sc_gather(idx2, tidx2, tbl3)
    return _tc_linear(g2, W.T, b.reshape(1, NLBL))
